# baseline (speedup 1.0000x reference)
import functools

import jax
import jax.numpy as jnp
from jax import lax
from jax.experimental import pallas as pl
from jax.experimental.pallas import tpu as pltpu
from jax.experimental.pallas import tpu_sc as plsc

N = 10000
R = 10240
E = 320000
DF = 128
NW = 32
CH = 128
NCH = 79
EW = NCH * CH
EP = NW * EW
NCHC = 2 * NCH
EWC = NCHC * CH
STR = R // 16
BR = 256
NBLK = R // BR

NB = 6
LA = 5


def _sc_mesh():
    return plsc.VectorSubcoreMesh(core_axis_name="c", subcore_axis_name="s")


def _run_chunk_loop(nch, g_start, g_wait, scatter):
    def step(b, j, do_gstart):
        g_wait(b, j)
        scatter(b, j)
        if do_gstart:
            g_start((b + LA) % NB, j + LA)

    for b in range(LA):
        g_start(b, b)
    for j in range(NB):
        step(j % NB, j, do_gstart=j + LA < nch)

    def block(q, carry):
        j0 = q * NB
        for b in range(NB):
            j = j0 + b
            g_wait(b, j)
            scatter(b, j)

            @pl.when(j + LA < nch)
            def _():
                g_start((b + LA) % NB, j + LA)
        return carry

    lax.fori_loop(1, nch // NB, block, 0)
    for j in range(NB * (nch // NB), nch):
        step(j % NB, j, do_gstart=j + LA < nch)


def _make_deg_kernel():
    @functools.partial(
        pl.kernel,
        mesh=_sc_mesh(),
        out_type=jax.ShapeDtypeStruct((2, R, 16), jnp.float32),
        scratch_types=[
            pltpu.VMEM((NCH, CH), jnp.int32),
            pltpu.VMEM((CH, 16), jnp.float32),
            pltpu.VMEM_SHARED((R, 16), jnp.float32),
        ],
        compiler_params=pltpu.CompilerParams(use_tc_tiling_on_sc=False),
    )
    def deg_kernel(dst_hbm, z_hbm, ones_hbm, out_hbm, dst_v, ones_v, acc):
        c = lax.axis_index("c")
        s = lax.axis_index("s")
        w = s * 2 + c
        pltpu.sync_copy(z_hbm.at[pl.ds(s * STR, STR)], acc.at[pl.ds(s * STR, STR)])
        pltpu.sync_copy(dst_hbm.at[w], dst_v)
        pltpu.sync_copy(ones_hbm, ones_v)
        plsc.subcore_barrier()

        def body(j, carry):
            pltpu.sync_copy(ones_v, acc.at[dst_v.at[j]], add=True)
            return carry

        lax.fori_loop(0, NCH, body, 0)
        plsc.subcore_barrier()
        pltpu.sync_copy(acc.at[pl.ds(s * STR, STR)],
                        out_hbm.at[c, pl.ds(s * STR, STR)])

    return deg_kernel


def _make_aggr_col_kernel(d2):
    @functools.partial(
        pl.kernel,
        mesh=_sc_mesh(),
        out_type=[jax.ShapeDtypeStruct((R, d2), jnp.float32),
                  jax.ShapeDtypeStruct((R, d2), jnp.float32)],
        scratch_types=[
            pltpu.VMEM((EWC,), jnp.int32),
            pltpu.VMEM((NCHC, CH), jnp.int32),
            [pltpu.VMEM((CH, d2), jnp.float32)] * NB,
            [pltpu.SemaphoreType.DMA] * NB,
            pltpu.VMEM_SHARED((R, d2), jnp.float32),
            pltpu.VMEM_SHARED((R, d2), jnp.float32),
        ],
        compiler_params=pltpu.CompilerParams(use_tc_tiling_on_sc=False),
    )
    def aggr_kernel(lo_hbm, hi_hbm, src_hbm, dst_hbm, z_hbm, out_lo, out_hi,
                    src_v, dst_v, rings, gsems, acc, hs_sh):
        c = lax.axis_index("c")
        s = lax.axis_index("s")

        @pl.when(c == 0)
        def _():
            pltpu.sync_copy(lo_hbm.at[pl.ds(s * STR, STR)],
                            hs_sh.at[pl.ds(s * STR, STR)])

        @pl.when(c == 1)
        def _():
            pltpu.sync_copy(hi_hbm.at[pl.ds(s * STR, STR)],
                            hs_sh.at[pl.ds(s * STR, STR)])

        pltpu.sync_copy(z_hbm.at[pl.ds(s * STR, STR)], acc.at[pl.ds(s * STR, STR)])
        pltpu.sync_copy(src_hbm.at[pl.ds(s * EWC, EWC)], src_v)
        pltpu.sync_copy(dst_hbm.at[s], dst_v)
        plsc.subcore_barrier()

        def g_start(b, j):
            pltpu.async_copy(hs_sh.at[src_v.at[pl.ds(j * CH, CH)]],
                             rings[b], gsems[b])

        def g_wait(b, j):
            pltpu.make_async_copy(hs_sh.at[src_v.at[pl.ds(j * CH, CH)]],
                                  rings[b], gsems[b]).wait()

        def scatter(b, j):
            pltpu.sync_copy(rings[b], acc.at[dst_v.at[j]], add=True)

        _run_chunk_loop(NCHC, g_start, g_wait, scatter)
        plsc.subcore_barrier()

        @pl.when(c == 0)
        def _():
            pltpu.sync_copy(acc.at[pl.ds(s * STR, STR)],
                            out_lo.at[pl.ds(s * STR, STR)])

        @pl.when(c == 1)
        def _():
            pltpu.sync_copy(acc.at[pl.ds(s * STR, STR)],
                            out_hi.at[pl.ds(s * STR, STR)])

    return aggr_kernel


def _make_aggr_edge_kernel(d):
    @functools.partial(
        pl.kernel,
        mesh=_sc_mesh(),
        out_type=jax.ShapeDtypeStruct((2, R, d), jnp.float32),
        scratch_types=[
            pltpu.VMEM((EW,), jnp.int32),
            pltpu.VMEM((NCH, CH), jnp.int32),
            [pltpu.VMEM((CH, d), jnp.float32)] * NB,
            [pltpu.SemaphoreType.DMA] * NB,
            pltpu.VMEM_SHARED((R, d), jnp.float32),
            pltpu.VMEM_SHARED((R, d), jnp.float32),
        ],
        compiler_params=pltpu.CompilerParams(use_tc_tiling_on_sc=False),
    )
    def aggr_kernel(hs_hbm, src_hbm, dst_hbm, z_hbm, out_hbm,
                    src_v, dst_v, rings, gsems, acc, hs_sh):
        c = lax.axis_index("c")
        s = lax.axis_index("s")
        w = s * 2 + c
        pltpu.sync_copy(hs_hbm.at[pl.ds(s * STR, STR)],
                        hs_sh.at[pl.ds(s * STR, STR)])
        pltpu.sync_copy(z_hbm.at[pl.ds(s * STR, STR)], acc.at[pl.ds(s * STR, STR)])
        pltpu.sync_copy(src_hbm.at[pl.ds(w * EW, EW)], src_v)
        pltpu.sync_copy(dst_hbm.at[w], dst_v)
        plsc.subcore_barrier()

        def g_start(b, j):
            pltpu.async_copy(hs_sh.at[src_v.at[pl.ds(j * CH, CH)]],
                             rings[b], gsems[b])

        def g_wait(b, j):
            pltpu.make_async_copy(hs_sh.at[src_v.at[pl.ds(j * CH, CH)]],
                                  rings[b], gsems[b]).wait()

        def scatter(b, j):
            pltpu.sync_copy(rings[b], acc.at[dst_v.at[j]], add=True)

        _run_chunk_loop(NCH, g_start, g_wait, scatter)
        plsc.subcore_barrier()
        pltpu.sync_copy(acc.at[pl.ds(s * STR, STR)],
                        out_hbm.at[c, pl.ds(s * STR, STR)])

    return aggr_kernel


def _tc_matmul1(x_pad, w1):
    def body(x_ref, w_ref, h_ref):
        h_ref[...] = jnp.dot(x_ref[...], w_ref[...],
                             preferred_element_type=jnp.float32)

    return pl.pallas_call(
        body,
        grid=(NBLK,),
        in_specs=[
            pl.BlockSpec((BR, DF), lambda i: (i, 0)),
            pl.BlockSpec((DF, 64), lambda i: (0, 0)),
        ],
        out_specs=pl.BlockSpec((BR, 64), lambda i: (i, 0)),
        out_shape=jax.ShapeDtypeStruct((R, 64), jnp.float32),
    )(x_pad, w1)


def _tc_scale1(h1, deg_parts):
    def body(d0_ref, d1_ref, h_ref, lo_ref, hi_ref, dinv_ref):
        deg = d0_ref[:, 0:1] + d1_ref[:, 0:1] + 1.0
        dinv = lax.rsqrt(deg)
        hs = h_ref[...] * dinv
        lo_ref[...] = hs[:, :32]
        hi_ref[...] = hs[:, 32:]
        dinv_ref[...] = dinv

    return pl.pallas_call(
        body,
        grid=(NBLK,),
        in_specs=[
            pl.BlockSpec((BR, 16), lambda i: (i, 0)),
            pl.BlockSpec((BR, 16), lambda i: (i, 0)),
            pl.BlockSpec((BR, 64), lambda i: (i, 0)),
        ],
        out_specs=[
            pl.BlockSpec((BR, 32), lambda i: (i, 0)),
            pl.BlockSpec((BR, 32), lambda i: (i, 0)),
            pl.BlockSpec((BR, 1), lambda i: (i, 0)),
        ],
        out_shape=[
            jax.ShapeDtypeStruct((R, 32), jnp.float32),
            jax.ShapeDtypeStruct((R, 32), jnp.float32),
            jax.ShapeDtypeStruct((R, 1), jnp.float32),
        ],
    )(deg_parts[0], deg_parts[1], h1)


def _tc_mid(alo, ahi, hlo, hhi, dinv, b, w_next, d, d_next, split_next):
    dh = d // 2
    dn2 = d_next // 2

    def body(alo_ref, ahi_ref, hlo_ref, hhi_ref, dinv_ref, b_ref, w_ref,
             f_ref, *hsn_refs):
        dinv = dinv_ref[...]
        f = jnp.concatenate(
            [alo_ref[...] + hlo_ref[...], ahi_ref[...] + hhi_ref[...]], axis=1)
        f = jnp.maximum(f * dinv + b_ref[...], 0.0)
        f_ref[...] = f
        hsn = jnp.dot(f, w_ref[...], preferred_element_type=jnp.float32) * dinv
        if split_next:
            hsn_refs[0][...] = hsn[:, :dn2]
            hsn_refs[1][...] = hsn[:, dn2:]
        else:
            hsn_refs[0][...] = hsn

    nout = [jax.ShapeDtypeStruct((R, dn2), jnp.float32)] * 2 if split_next \
        else [jax.ShapeDtypeStruct((R, d_next), jnp.float32)]
    nspec = [pl.BlockSpec((BR, dn2), lambda i: (i, 0))] * 2 if split_next \
        else [pl.BlockSpec((BR, d_next), lambda i: (i, 0))]

    return pl.pallas_call(
        body,
        grid=(NBLK,),
        in_specs=[
            pl.BlockSpec((BR, dh), lambda i: (i, 0)),
            pl.BlockSpec((BR, dh), lambda i: (i, 0)),
            pl.BlockSpec((BR, dh), lambda i: (i, 0)),
            pl.BlockSpec((BR, dh), lambda i: (i, 0)),
            pl.BlockSpec((BR, 1), lambda i: (i, 0)),
            pl.BlockSpec((1, d), lambda i: (0, 0)),
            pl.BlockSpec((d, d_next), lambda i: (0, 0)),
        ],
        out_specs=[pl.BlockSpec((BR, d), lambda i: (i, 0))] + nspec,
        out_shape=[jax.ShapeDtypeStruct((R, d), jnp.float32)] + nout,
    )(alo, ahi, hlo, hhi, dinv, b, w_next)


def _tc_last(parts, hs3, dinv, b3, f1, f2, wfc1, wfc2, wfc3, bfc):
    def body(p0_ref, p1_ref, hs_ref, dinv_ref, b3_ref, f1_ref, f2_ref,
             w1_ref, w2_ref, w3_ref, bfc_ref, out_ref):
        f3 = (p0_ref[...] + p1_ref[...] + hs_ref[...]) * dinv_ref[...] + b3_ref[...]
        f3 = jnp.maximum(f3, 0.0)
        acc = jnp.dot(f1_ref[...], w1_ref[...], preferred_element_type=jnp.float32)
        acc += jnp.dot(f2_ref[...], w2_ref[...], preferred_element_type=jnp.float32)
        acc += jnp.dot(f3, w3_ref[...], preferred_element_type=jnp.float32)
        out_ref[...] = jnp.maximum(acc + bfc_ref[...], 0.0)

    return pl.pallas_call(
        body,
        grid=(NBLK,),
        in_specs=[
            pl.BlockSpec((BR, 16), lambda i: (i, 0)),
            pl.BlockSpec((BR, 16), lambda i: (i, 0)),
            pl.BlockSpec((BR, 16), lambda i: (i, 0)),
            pl.BlockSpec((BR, 1), lambda i: (i, 0)),
            pl.BlockSpec((1, 16), lambda i: (0, 0)),
            pl.BlockSpec((BR, 64), lambda i: (i, 0)),
            pl.BlockSpec((BR, 32), lambda i: (i, 0)),
            pl.BlockSpec((64, 16), lambda i: (0, 0)),
            pl.BlockSpec((32, 16), lambda i: (0, 0)),
            pl.BlockSpec((16, 16), lambda i: (0, 0)),
            pl.BlockSpec((1, 16), lambda i: (0, 0)),
        ],
        out_specs=pl.BlockSpec((BR, 16), lambda i: (i, 0)),
        out_shape=jax.ShapeDtypeStruct((R, 16), jnp.float32),
    )(parts[0], parts[1], hs3, dinv, b3, f1, f2, wfc1, wfc2, wfc3, bfc)


_deg_kernel = _make_deg_kernel()
_aggr64 = _make_aggr_col_kernel(32)
_aggr32 = _make_aggr_col_kernel(16)
_aggr16 = _make_aggr_edge_kernel(16)


def kernel(edges, features, W1, b1, W2, b2, W3, b3, Wfc, bfc):
    edges = edges.astype(jnp.int32)
    pad = jnp.full((EP - E,), N, jnp.int32)
    src = jnp.concatenate([edges[0], pad])
    dst = jnp.concatenate([edges[1], pad])
    dst32 = dst.reshape(NW, NCH, CH)
    dst16 = dst.reshape(16, NCHC, CH)

    x_pad = jnp.zeros((R, DF), jnp.float32).at[:N].set(features)
    z32 = jnp.zeros((R, 32), jnp.float32)
    z16 = jnp.zeros((R, 16), jnp.float32)
    ones128 = jnp.ones((CH, 16), jnp.float32)

    deg_parts = _deg_kernel(dst32, z16, ones128)
    h1 = _tc_matmul1(x_pad, W1)
    h1lo, h1hi, dinv = _tc_scale1(h1, deg_parts)

    a1lo, a1hi = _aggr64(h1lo, h1hi, src, dst16, z32)
    f1, h2lo, h2hi = _tc_mid(a1lo, a1hi, h1lo, h1hi, dinv,
                             b1.reshape(1, 64), W2, 64, 32, split_next=True)

    a2lo, a2hi = _aggr32(h2lo, h2hi, src, dst16, z16)
    f2, hs3 = _tc_mid(a2lo, a2hi, h2lo, h2hi, dinv,
                      b2.reshape(1, 32), W3, 32, 16, split_next=False)

    parts3 = _aggr16(hs3, src, dst32, z16)
    out = _tc_last(parts3, hs3, dinv, b3.reshape(1, 16), f1, f2,
                   Wfc[:64], Wfc[64:96], Wfc[96:112], bfc.reshape(1, 16))
    return out[:N]


# speedup vs baseline: 1.0248x; 1.0248x over previous
import functools

import jax
import jax.numpy as jnp
from jax import lax
from jax.experimental import pallas as pl
from jax.experimental.pallas import tpu as pltpu
from jax.experimental.pallas import tpu_sc as plsc

N = 10000
R = 10240
E = 320000
DF = 128
NW = 32
CH = 128
NCH = 79
EW = NCH * CH
EP = NW * EW
NCHC = 2 * NCH
EWC = NCHC * CH
STR = R // 16
BR = 256
NBLK = R // BR

NB = 6
LA = 5


def _sc_mesh():
    return plsc.VectorSubcoreMesh(core_axis_name="c", subcore_axis_name="s")


def _run_chunk_loop(nch, g_start, g_wait, scatter):
    def step(b, j, do_gstart):
        g_wait(b, j)
        scatter(b, j)
        if do_gstart:
            g_start((b + LA) % NB, j + LA)

    for b in range(LA):
        g_start(b, b)
    for j in range(NB):
        step(j % NB, j, do_gstart=j + LA < nch)

    def block(q, carry):
        j0 = q * NB
        for b in range(NB):
            j = j0 + b
            g_wait(b, j)
            scatter(b, j)

            @pl.when(j + LA < nch)
            def _():
                g_start((b + LA) % NB, j + LA)
        return carry

    lax.fori_loop(1, nch // NB, block, 0)
    for j in range(NB * (nch // NB), nch):
        step(j % NB, j, do_gstart=j + LA < nch)


def _make_deg_kernel():
    @functools.partial(
        pl.kernel,
        mesh=_sc_mesh(),
        out_type=jax.ShapeDtypeStruct((2, R, 16), jnp.float32),
        scratch_types=[
            pltpu.VMEM((NCH, CH), jnp.int32),
            pltpu.VMEM((CH, 16), jnp.float32),
            pltpu.VMEM_SHARED((R, 16), jnp.float32),
        ],
        compiler_params=pltpu.CompilerParams(use_tc_tiling_on_sc=False),
    )
    def deg_kernel(dst_hbm, z_hbm, ones_hbm, out_hbm, dst_v, ones_v, acc):
        c = lax.axis_index("c")
        s = lax.axis_index("s")
        w = s * 2 + c
        pltpu.sync_copy(z_hbm.at[pl.ds(s * STR, STR)], acc.at[pl.ds(s * STR, STR)])
        pltpu.sync_copy(dst_hbm.at[w], dst_v)
        pltpu.sync_copy(ones_hbm, ones_v)
        plsc.subcore_barrier()

        def body(j, carry):
            pltpu.sync_copy(ones_v, acc.at[dst_v.at[j]], add=True)
            return carry

        lax.fori_loop(0, NCH, body, 0)
        plsc.subcore_barrier()
        pltpu.sync_copy(acc.at[pl.ds(s * STR, STR)],
                        out_hbm.at[c, pl.ds(s * STR, STR)])

    return deg_kernel


def _make_aggr_col_kernel(d2):
    @functools.partial(
        pl.kernel,
        mesh=_sc_mesh(),
        out_type=[jax.ShapeDtypeStruct((R, d2), jnp.float32),
                  jax.ShapeDtypeStruct((R, d2), jnp.float32)],
        scratch_types=[
            pltpu.VMEM((EWC,), jnp.int32),
            pltpu.VMEM((NCHC, CH), jnp.int32),
            [pltpu.VMEM((CH, d2), jnp.float32)] * NB,
            [pltpu.SemaphoreType.DMA] * NB,
            pltpu.VMEM_SHARED((R, d2), jnp.float32),
            pltpu.VMEM_SHARED((R, d2), jnp.float32),
        ],
        compiler_params=pltpu.CompilerParams(use_tc_tiling_on_sc=False),
    )
    def aggr_kernel(lo_hbm, hi_hbm, src_hbm, dst_hbm, z_hbm, out_lo, out_hi,
                    src_v, dst_v, rings, gsems, acc, hs_sh):
        c = lax.axis_index("c")
        s = lax.axis_index("s")

        @pl.when(c == 0)
        def _():
            pltpu.sync_copy(lo_hbm.at[pl.ds(s * STR, STR)],
                            hs_sh.at[pl.ds(s * STR, STR)])

        @pl.when(c == 1)
        def _():
            pltpu.sync_copy(hi_hbm.at[pl.ds(s * STR, STR)],
                            hs_sh.at[pl.ds(s * STR, STR)])

        pltpu.sync_copy(z_hbm.at[pl.ds(s * STR, STR)], acc.at[pl.ds(s * STR, STR)])
        pltpu.sync_copy(src_hbm.at[pl.ds(s * EWC, EWC)], src_v)
        pltpu.sync_copy(dst_hbm.at[s], dst_v)
        plsc.subcore_barrier()

        def g_start(b, j):
            pltpu.async_copy(hs_sh.at[src_v.at[pl.ds(j * CH, CH)]],
                             rings[b], gsems[b])

        def g_wait(b, j):
            pltpu.make_async_copy(hs_sh.at[src_v.at[pl.ds(j * CH, CH)]],
                                  rings[b], gsems[b]).wait()

        def scatter(b, j):
            pltpu.sync_copy(rings[b], acc.at[dst_v.at[j]], add=True)

        _run_chunk_loop(NCHC, g_start, g_wait, scatter)
        plsc.subcore_barrier()

        @pl.when(c == 0)
        def _():
            pltpu.sync_copy(acc.at[pl.ds(s * STR, STR)],
                            out_lo.at[pl.ds(s * STR, STR)])

        @pl.when(c == 1)
        def _():
            pltpu.sync_copy(acc.at[pl.ds(s * STR, STR)],
                            out_hi.at[pl.ds(s * STR, STR)])

    return aggr_kernel


def _make_aggr_edge_kernel(d):
    @functools.partial(
        pl.kernel,
        mesh=_sc_mesh(),
        out_type=jax.ShapeDtypeStruct((2, R, d), jnp.float32),
        scratch_types=[
            pltpu.VMEM((EW,), jnp.int32),
            pltpu.VMEM((NCH, CH), jnp.int32),
            [pltpu.VMEM((CH, d), jnp.float32)] * NB,
            [pltpu.SemaphoreType.DMA] * NB,
            pltpu.VMEM_SHARED((R, d), jnp.float32),
            pltpu.VMEM_SHARED((R, d), jnp.float32),
        ],
        compiler_params=pltpu.CompilerParams(use_tc_tiling_on_sc=False),
    )
    def aggr_kernel(hs_hbm, src_hbm, dst_hbm, z_hbm, out_hbm,
                    src_v, dst_v, rings, gsems, acc, hs_sh):
        c = lax.axis_index("c")
        s = lax.axis_index("s")
        w = s * 2 + c
        pltpu.sync_copy(hs_hbm.at[pl.ds(s * STR, STR)],
                        hs_sh.at[pl.ds(s * STR, STR)])
        pltpu.sync_copy(z_hbm.at[pl.ds(s * STR, STR)], acc.at[pl.ds(s * STR, STR)])
        pltpu.sync_copy(src_hbm.at[pl.ds(w * EW, EW)], src_v)
        pltpu.sync_copy(dst_hbm.at[w], dst_v)
        plsc.subcore_barrier()

        def g_start(b, j):
            pltpu.async_copy(hs_sh.at[src_v.at[pl.ds(j * CH, CH)]],
                             rings[b], gsems[b])

        def g_wait(b, j):
            pltpu.make_async_copy(hs_sh.at[src_v.at[pl.ds(j * CH, CH)]],
                                  rings[b], gsems[b]).wait()

        def scatter(b, j):
            pltpu.sync_copy(rings[b], acc.at[dst_v.at[j]], add=True)

        _run_chunk_loop(NCH, g_start, g_wait, scatter)
        plsc.subcore_barrier()
        pltpu.sync_copy(acc.at[pl.ds(s * STR, STR)],
                        out_hbm.at[c, pl.ds(s * STR, STR)])

    return aggr_kernel


def _tc_first(x_pad, w1, deg_parts):
    def body(d0_ref, d1_ref, x_ref, w_ref, lo_ref, hi_ref, dinv_ref):
        deg = d0_ref[:, 0:1] + d1_ref[:, 0:1] + 1.0
        dinv = lax.rsqrt(deg)
        h = jnp.dot(x_ref[...], w_ref[...], preferred_element_type=jnp.float32)
        hs = h * dinv
        lo_ref[...] = hs[:, :32]
        hi_ref[...] = hs[:, 32:]
        dinv_ref[...] = dinv

    return pl.pallas_call(
        body,
        grid=(NBLK,),
        in_specs=[
            pl.BlockSpec((BR, 16), lambda i: (i, 0)),
            pl.BlockSpec((BR, 16), lambda i: (i, 0)),
            pl.BlockSpec((BR, DF), lambda i: (i, 0)),
            pl.BlockSpec((DF, 64), lambda i: (0, 0)),
        ],
        out_specs=[
            pl.BlockSpec((BR, 32), lambda i: (i, 0)),
            pl.BlockSpec((BR, 32), lambda i: (i, 0)),
            pl.BlockSpec((BR, 1), lambda i: (i, 0)),
        ],
        out_shape=[
            jax.ShapeDtypeStruct((R, 32), jnp.float32),
            jax.ShapeDtypeStruct((R, 32), jnp.float32),
            jax.ShapeDtypeStruct((R, 1), jnp.float32),
        ],
    )(deg_parts[0], deg_parts[1], x_pad, w1)


def _tc_mid(alo, ahi, hlo, hhi, dinv, b, w_next, d, d_next, split_next):
    dh = d // 2
    dn2 = d_next // 2

    def body(alo_ref, ahi_ref, hlo_ref, hhi_ref, dinv_ref, b_ref, w_ref,
             f_ref, *hsn_refs):
        dinv = dinv_ref[...]
        f = jnp.concatenate(
            [alo_ref[...] + hlo_ref[...], ahi_ref[...] + hhi_ref[...]], axis=1)
        f = jnp.maximum(f * dinv + b_ref[...], 0.0)
        f_ref[...] = f
        hsn = jnp.dot(f, w_ref[...], preferred_element_type=jnp.float32) * dinv
        if split_next:
            hsn_refs[0][...] = hsn[:, :dn2]
            hsn_refs[1][...] = hsn[:, dn2:]
        else:
            hsn_refs[0][...] = hsn

    nout = [jax.ShapeDtypeStruct((R, dn2), jnp.float32)] * 2 if split_next \
        else [jax.ShapeDtypeStruct((R, d_next), jnp.float32)]
    nspec = [pl.BlockSpec((BR, dn2), lambda i: (i, 0))] * 2 if split_next \
        else [pl.BlockSpec((BR, d_next), lambda i: (i, 0))]

    return pl.pallas_call(
        body,
        grid=(NBLK,),
        in_specs=[
            pl.BlockSpec((BR, dh), lambda i: (i, 0)),
            pl.BlockSpec((BR, dh), lambda i: (i, 0)),
            pl.BlockSpec((BR, dh), lambda i: (i, 0)),
            pl.BlockSpec((BR, dh), lambda i: (i, 0)),
            pl.BlockSpec((BR, 1), lambda i: (i, 0)),
            pl.BlockSpec((1, d), lambda i: (0, 0)),
            pl.BlockSpec((d, d_next), lambda i: (0, 0)),
        ],
        out_specs=[pl.BlockSpec((BR, d), lambda i: (i, 0))] + nspec,
        out_shape=[jax.ShapeDtypeStruct((R, d), jnp.float32)] + nout,
    )(alo, ahi, hlo, hhi, dinv, b, w_next)


def _tc_last(parts, hs3, dinv, b3, f1, f2, wfc1, wfc2, wfc3, bfc):
    def body(p0_ref, p1_ref, hs_ref, dinv_ref, b3_ref, f1_ref, f2_ref,
             w1_ref, w2_ref, w3_ref, bfc_ref, out_ref):
        f3 = (p0_ref[...] + p1_ref[...] + hs_ref[...]) * dinv_ref[...] + b3_ref[...]
        f3 = jnp.maximum(f3, 0.0)
        acc = jnp.dot(f1_ref[...], w1_ref[...], preferred_element_type=jnp.float32)
        acc += jnp.dot(f2_ref[...], w2_ref[...], preferred_element_type=jnp.float32)
        acc += jnp.dot(f3, w3_ref[...], preferred_element_type=jnp.float32)
        out_ref[...] = jnp.maximum(acc + bfc_ref[...], 0.0)

    return pl.pallas_call(
        body,
        grid=(NBLK,),
        in_specs=[
            pl.BlockSpec((BR, 16), lambda i: (i, 0)),
            pl.BlockSpec((BR, 16), lambda i: (i, 0)),
            pl.BlockSpec((BR, 16), lambda i: (i, 0)),
            pl.BlockSpec((BR, 1), lambda i: (i, 0)),
            pl.BlockSpec((1, 16), lambda i: (0, 0)),
            pl.BlockSpec((BR, 64), lambda i: (i, 0)),
            pl.BlockSpec((BR, 32), lambda i: (i, 0)),
            pl.BlockSpec((64, 16), lambda i: (0, 0)),
            pl.BlockSpec((32, 16), lambda i: (0, 0)),
            pl.BlockSpec((16, 16), lambda i: (0, 0)),
            pl.BlockSpec((1, 16), lambda i: (0, 0)),
        ],
        out_specs=pl.BlockSpec((BR, 16), lambda i: (i, 0)),
        out_shape=jax.ShapeDtypeStruct((R, 16), jnp.float32),
    )(parts[0], parts[1], hs3, dinv, b3, f1, f2, wfc1, wfc2, wfc3, bfc)


_deg_kernel = _make_deg_kernel()
_aggr64 = _make_aggr_col_kernel(32)
_aggr32 = _make_aggr_col_kernel(16)
_aggr16 = _make_aggr_edge_kernel(16)


def kernel(edges, features, W1, b1, W2, b2, W3, b3, Wfc, bfc):
    edges = edges.astype(jnp.int32)
    pad = jnp.full((EP - E,), N, jnp.int32)
    src = jnp.concatenate([edges[0], pad])
    dst = jnp.concatenate([edges[1], pad])
    dst32 = dst.reshape(NW, NCH, CH)
    dst16 = dst.reshape(16, NCHC, CH)

    x_pad = jnp.zeros((R, DF), jnp.float32).at[:N].set(features)
    z32 = jnp.zeros((R, 32), jnp.float32)
    z16 = jnp.zeros((R, 16), jnp.float32)
    ones128 = jnp.ones((CH, 16), jnp.float32)

    deg_parts = _deg_kernel(dst32, z16, ones128)
    h1lo, h1hi, dinv = _tc_first(x_pad, W1, deg_parts)

    a1lo, a1hi = _aggr64(h1lo, h1hi, src, dst16, z32)
    f1, h2lo, h2hi = _tc_mid(a1lo, a1hi, h1lo, h1hi, dinv,
                             b1.reshape(1, 64), W2, 64, 32, split_next=True)

    a2lo, a2hi = _aggr32(h2lo, h2hi, src, dst16, z16)
    f2, hs3 = _tc_mid(a2lo, a2hi, h2lo, h2hi, dinv,
                      b2.reshape(1, 32), W3, 32, 16, split_next=False)

    parts3 = _aggr16(hs3, src, dst32, z16)
    out = _tc_last(parts3, hs3, dinv, b3.reshape(1, 16), f1, f2,
                   Wfc[:64], Wfc[64:96], Wfc[96:112], bfc.reshape(1, 16))
    return out[:N]
